# baseline (speedup 1.0000x reference)
import functools

import jax
import jax.numpy as jnp
from jax import lax
from jax.experimental import pallas as pl
from jax.experimental.pallas import tpu as pltpu
from jax.experimental.pallas import tpu_sc as plsc

BETA = 0.95
THRESH = 1.0

NC = 2
NS = 16
TILES = NC * NS
CH = 128


def _sc_mesh():
  return plsc.VectorSubcoreMesh(core_axis_name="c", subcore_axis_name="s")


_SC_PARAMS = pltpu.CompilerParams(use_tc_tiling_on_sc=False)


def _make_deg_kernel(np_, nch):
  rows = np_ // NS

  @functools.partial(
      pl.kernel,
      out_type=jax.ShapeDtypeStruct((NC, np_, 16), jnp.float32),
      mesh=_sc_mesh(),
      compiler_params=_SC_PARAMS,
      scratch_types=[
          pltpu.VMEM((nch, CH), jnp.int32),
          pltpu.VMEM((CH, 16), jnp.float32),
          pltpu.VMEM_SHARED((np_, 16), jnp.float32),
      ],
  )
  def deg_kernel(dst_hbm, ones_hbm, zeros_hbm, out_hbm, dst_v, ones_v, deg_sp):
    c = lax.axis_index("c")
    s = lax.axis_index("s")
    wid = c * NS + s
    pltpu.sync_copy(dst_hbm.at[pl.ds(wid * nch, nch)], dst_v)
    pltpu.sync_copy(ones_hbm, ones_v)
    pltpu.sync_copy(zeros_hbm, deg_sp.at[pl.ds(s * rows, rows)])
    plsc.subcore_barrier()

    @pl.loop(0, nch)
    def _(j):
      pltpu.sync_copy(ones_v, deg_sp.at[dst_v.at[j]], add=True)

    plsc.subcore_barrier()
    pltpu.sync_copy(deg_sp.at[pl.ds(s * rows, rows)],
                    out_hbm.at[c, pl.ds(s * rows, rows)])

  return deg_kernel


def _make_edge_kernel(n, np_, d, nch):
  rows = np_ // NS
  half = nch // 2
  full_tiles = n // rows
  rem = n - full_tiles * rows

  @functools.partial(
      pl.kernel,
      out_type=jax.ShapeDtypeStruct((NC, np_, d), jnp.float32),
      mesh=_sc_mesh(),
      compiler_params=_SC_PARAMS,
      scratch_types=[
          pltpu.VMEM((half, CH), jnp.int32),
          pltpu.VMEM((half, CH), jnp.int32),
          pltpu.VMEM((CH, d), jnp.float32),
          pltpu.VMEM((CH, d), jnp.float32),
          pltpu.VMEM_SHARED((np_, d), jnp.float32),
          pltpu.SemaphoreType.DMA,
          pltpu.SemaphoreType.DMA,
      ],
  )
  def edge_kernel(y_hbm, src_hbm, dst_hbm, zeros_hbm, out_hbm,
                  src_v, dst_v, rows_a, rows_b, acc_sp, sem, sem2):
    c = lax.axis_index("c")
    s = lax.axis_index("s")
    wid = c * NS + s

    @pl.when(jnp.logical_and(c == 0, s < full_tiles))
    def _():
      pltpu.sync_copy(y_hbm.at[pl.ds(s * rows, rows)],
                      acc_sp.at[pl.ds(s * rows, rows)])

    if rem > 0:
      @pl.when(jnp.logical_and(c == 0, s == full_tiles))
      def _():
        pltpu.sync_copy(y_hbm.at[pl.ds(full_tiles * rows, rem)],
                        acc_sp.at[pl.ds(full_tiles * rows, rem)])

    @pl.when(c != 0)
    def _():
      pltpu.sync_copy(zeros_hbm, acc_sp.at[pl.ds(s * rows, rows)])

    plsc.subcore_barrier()

    bufs = (rows_a, rows_b)
    for h in range(2):
      base = wid * nch + h * half
      pltpu.sync_copy(src_hbm.at[pl.ds(base, half)], src_v)
      pltpu.sync_copy(dst_hbm.at[pl.ds(base, half)], dst_v)
      pltpu.async_copy(y_hbm.at[src_v.at[0]], rows_a, sem)

      @pl.loop(0, half, step=2)
      def _(j):
        for k in range(2):
          cur, nxt = bufs[k], bufs[1 - k]
          jj = j + k
          pltpu.make_async_copy(y_hbm.at[src_v.at[jj]], cur, sem).wait()

          @pl.when(jj + 1 < half)
          def _():
            @pl.when(jj >= 1)
            def _():
              pltpu.make_async_copy(
                  nxt, acc_sp.at[dst_v.at[jj]], sem2).wait()

            pltpu.async_copy(y_hbm.at[src_v.at[jj + 1]], nxt, sem)

          pltpu.async_copy(cur, acc_sp.at[dst_v.at[jj]], sem2, add=True)

      pltpu.make_async_copy(rows_a, acc_sp.at[dst_v.at[0]], sem2).wait()
      pltpu.make_async_copy(rows_b, acc_sp.at[dst_v.at[0]], sem2).wait()

    plsc.subcore_barrier()
    pltpu.sync_copy(acc_sp.at[pl.ds(s * rows, rows)],
                    out_hbm.at[c, pl.ds(s * rows, rows)])

  return edge_kernel


def _mm_scale_body(x_ref, w_ref, deg_ref, y_ref):
  xw = jnp.dot(x_ref[...], w_ref[...], preferred_element_type=jnp.float32)
  deg = deg_ref[0, :, :1] + deg_ref[1, :, :1] + 1.0
  y_ref[...] = xw * lax.rsqrt(deg)


def _mm_scale_call(x, w, deg):
  n, d_in = x.shape
  d_out = w.shape[1]
  blk = 1000
  return pl.pallas_call(
      _mm_scale_body,
      grid=(n // blk,),
      in_specs=[
          pl.BlockSpec((blk, d_in), lambda i: (i, 0)),
          pl.BlockSpec((d_in, d_out), lambda i: (0, 0)),
          pl.BlockSpec((NC, blk, 16), lambda i: (0, i, 0)),
      ],
      out_specs=pl.BlockSpec((blk, d_out), lambda i: (i, 0)),
      out_shape=jax.ShapeDtypeStruct((n, d_out), jnp.float32),
  )(x, w, deg)


def _final_body(acc_ref, deg_ref, mem_ref, b_ref, spk_ref, mem_out_ref):
  deg = deg_ref[0, :, :1] + deg_ref[1, :, :1] + 1.0
  dis = lax.rsqrt(deg)
  out = (acc_ref[0] + acc_ref[1]) * dis + b_ref[...]
  mem = mem_ref[...]
  reset = (mem - THRESH > 0).astype(jnp.float32)
  mem_new = BETA * mem + out - reset * THRESH
  spk_ref[...] = (mem_new - THRESH > 0).astype(jnp.float32)
  mem_out_ref[...] = mem_new


def _final_call(acc, deg, mem, b):
  n, d = mem.shape
  blk = 1000
  grid = (n // blk,)
  return pl.pallas_call(
      _final_body,
      grid=grid,
      in_specs=[
          pl.BlockSpec((NC, blk, d), lambda i: (0, i, 0)),
          pl.BlockSpec((NC, blk, 16), lambda i: (0, i, 0)),
          pl.BlockSpec((blk, d), lambda i: (i, 0)),
          pl.BlockSpec((1, d), lambda i: (0, 0)),
      ],
      out_specs=[
          pl.BlockSpec((blk, d), lambda i: (i, 0)),
          pl.BlockSpec((blk, d), lambda i: (i, 0)),
      ],
      out_shape=[
          jax.ShapeDtypeStruct((n, d), jnp.float32),
          jax.ShapeDtypeStruct((n, d), jnp.float32),
      ],
  )(acc, deg, mem, b)


def kernel(x, edge_index, mem, W, b):
  n, d_in = x.shape
  d_out = W.shape[1]
  e = edge_index.shape[1]

  grain = TILES * CH * 2
  ep = ((e + grain - 1) // grain) * grain
  nch = ep // (TILES * CH)
  rows = ((n // NS) // 8 + 1) * 8
  np_ = rows * NS
  assert np_ > n and n % 8 == 0

  ei = edge_index.astype(jnp.int32)
  pad = ep - e
  pad_ar = jnp.arange(pad, dtype=jnp.int32)
  src_p = jnp.concatenate([ei[0], pad_ar % n])
  dst_p = jnp.concatenate([ei[1], n + pad_ar % (np_ - n)])
  src2 = src_p.reshape(TILES * nch, CH)
  dst2 = dst_p.reshape(TILES * nch, CH)
  ones16 = jnp.ones((CH, 16), jnp.float32)
  zeros16 = jnp.zeros((rows, 16), jnp.float32)
  zeros_d = jnp.zeros((rows, d_out), jnp.float32)

  deg = _make_deg_kernel(np_, nch)(dst2, ones16, zeros16)
  y = _mm_scale_call(x, W, deg)
  acc = _make_edge_kernel(n, np_, d_out, nch)(y, src2, dst2, zeros_d)
  spk, mem_new = _final_call(acc, deg, mem, b.reshape(1, d_out))
  return (spk, mem_new)


# speedup vs baseline: 1.0165x; 1.0165x over previous
import functools

import jax
import jax.numpy as jnp
from jax import lax
from jax.experimental import pallas as pl
from jax.experimental.pallas import tpu as pltpu
from jax.experimental.pallas import tpu_sc as plsc

BETA = 0.95
THRESH = 1.0

NC = 2
NS = 16
TILES = NC * NS
CH = 128


def _sc_mesh():
  return plsc.VectorSubcoreMesh(core_axis_name="c", subcore_axis_name="s")


_SC_PARAMS = pltpu.CompilerParams(use_tc_tiling_on_sc=False)


def _make_deg_kernel(np_, nch):
  rows = np_ // NS

  @functools.partial(
      pl.kernel,
      out_type=jax.ShapeDtypeStruct((NC, np_, 16), jnp.float32),
      mesh=_sc_mesh(),
      compiler_params=_SC_PARAMS,
      scratch_types=[
          pltpu.VMEM((nch, CH), jnp.int32),
          pltpu.VMEM((CH, 16), jnp.float32),
          pltpu.VMEM_SHARED((np_, 16), jnp.float32),
          pltpu.SemaphoreType.DMA,
      ],
  )
  def deg_kernel(dst_hbm, ones_hbm, zeros_hbm, out_hbm, dst_v, ones_v, deg_sp,
                 sem):
    c = lax.axis_index("c")
    s = lax.axis_index("s")
    wid = c * NS + s
    pltpu.sync_copy(dst_hbm.at[pl.ds(wid * nch, nch)], dst_v)
    pltpu.sync_copy(ones_hbm, ones_v)
    pltpu.sync_copy(zeros_hbm, deg_sp.at[pl.ds(s * rows, rows)])
    plsc.subcore_barrier()

    @pl.loop(0, nch)
    def _(j):
      pltpu.async_copy(ones_v, deg_sp.at[dst_v.at[j]], sem, add=True)

    @pl.loop(0, nch)
    def _(j):
      pltpu.make_async_copy(ones_v, deg_sp.at[dst_v.at[0]], sem).wait()

    plsc.subcore_barrier()
    pltpu.sync_copy(deg_sp.at[pl.ds(s * rows, rows)],
                    out_hbm.at[c, pl.ds(s * rows, rows)])

  return deg_kernel


def _make_edge_kernel(n, np_, d, nch):
  rows = np_ // NS
  half = nch // 2
  full_tiles = n // rows
  rem = n - full_tiles * rows

  @functools.partial(
      pl.kernel,
      out_type=jax.ShapeDtypeStruct((NC, np_, d), jnp.float32),
      mesh=_sc_mesh(),
      compiler_params=_SC_PARAMS,
      scratch_types=[
          pltpu.VMEM((half, CH), jnp.int32),
          pltpu.VMEM((half, CH), jnp.int32),
          pltpu.VMEM((CH, d), jnp.float32),
          pltpu.VMEM((CH, d), jnp.float32),
          pltpu.VMEM_SHARED((np_, d), jnp.float32),
          pltpu.SemaphoreType.DMA,
          pltpu.SemaphoreType.DMA,
      ],
  )
  def edge_kernel(y_hbm, src_hbm, dst_hbm, zeros_hbm, out_hbm,
                  src_v, dst_v, rows_a, rows_b, acc_sp, sem, sem2):
    c = lax.axis_index("c")
    s = lax.axis_index("s")
    wid = c * NS + s

    @pl.when(jnp.logical_and(c == 0, s < full_tiles))
    def _():
      pltpu.sync_copy(y_hbm.at[pl.ds(s * rows, rows)],
                      acc_sp.at[pl.ds(s * rows, rows)])

    if rem > 0:
      @pl.when(jnp.logical_and(c == 0, s == full_tiles))
      def _():
        pltpu.sync_copy(y_hbm.at[pl.ds(full_tiles * rows, rem)],
                        acc_sp.at[pl.ds(full_tiles * rows, rem)])

    @pl.when(c != 0)
    def _():
      pltpu.sync_copy(zeros_hbm, acc_sp.at[pl.ds(s * rows, rows)])

    plsc.subcore_barrier()

    bufs = (rows_a, rows_b)
    for h in range(2):
      base = wid * nch + h * half
      pltpu.sync_copy(src_hbm.at[pl.ds(base, half)], src_v)
      pltpu.sync_copy(dst_hbm.at[pl.ds(base, half)], dst_v)
      pltpu.async_copy(y_hbm.at[src_v.at[0]], rows_a, sem)

      @pl.loop(0, half, step=2)
      def _(j):
        for k in range(2):
          cur, nxt = bufs[k], bufs[1 - k]
          jj = j + k
          pltpu.make_async_copy(y_hbm.at[src_v.at[jj]], cur, sem).wait()

          @pl.when(jj + 1 < half)
          def _():
            @pl.when(jj >= 1)
            def _():
              pltpu.make_async_copy(
                  nxt, acc_sp.at[dst_v.at[jj]], sem2).wait()

            pltpu.async_copy(y_hbm.at[src_v.at[jj + 1]], nxt, sem)

          pltpu.async_copy(cur, acc_sp.at[dst_v.at[jj]], sem2, add=True)

      pltpu.make_async_copy(rows_a, acc_sp.at[dst_v.at[0]], sem2).wait()
      pltpu.make_async_copy(rows_b, acc_sp.at[dst_v.at[0]], sem2).wait()

    plsc.subcore_barrier()
    pltpu.sync_copy(acc_sp.at[pl.ds(s * rows, rows)],
                    out_hbm.at[c, pl.ds(s * rows, rows)])

  return edge_kernel


def _mm_scale_body(x_ref, w_ref, deg_ref, y_ref):
  xw = jnp.dot(x_ref[...], w_ref[...], preferred_element_type=jnp.float32)
  deg = deg_ref[0, :, :1] + deg_ref[1, :, :1] + 1.0
  y_ref[...] = xw * lax.rsqrt(deg)


def _mm_scale_call(x, w, deg):
  n, d_in = x.shape
  d_out = w.shape[1]
  blk = 1000
  return pl.pallas_call(
      _mm_scale_body,
      grid=(n // blk,),
      in_specs=[
          pl.BlockSpec((blk, d_in), lambda i: (i, 0)),
          pl.BlockSpec((d_in, d_out), lambda i: (0, 0)),
          pl.BlockSpec((NC, blk, 16), lambda i: (0, i, 0)),
      ],
      out_specs=pl.BlockSpec((blk, d_out), lambda i: (i, 0)),
      out_shape=jax.ShapeDtypeStruct((n, d_out), jnp.float32),
  )(x, w, deg)


def _final_body(acc_ref, deg_ref, mem_ref, b_ref, spk_ref, mem_out_ref):
  deg = deg_ref[0, :, :1] + deg_ref[1, :, :1] + 1.0
  dis = lax.rsqrt(deg)
  out = (acc_ref[0] + acc_ref[1]) * dis + b_ref[...]
  mem = mem_ref[...]
  reset = (mem - THRESH > 0).astype(jnp.float32)
  mem_new = BETA * mem + out - reset * THRESH
  spk_ref[...] = (mem_new - THRESH > 0).astype(jnp.float32)
  mem_out_ref[...] = mem_new


def _final_call(acc, deg, mem, b):
  n, d = mem.shape
  blk = 1000
  grid = (n // blk,)
  return pl.pallas_call(
      _final_body,
      grid=grid,
      in_specs=[
          pl.BlockSpec((NC, blk, d), lambda i: (0, i, 0)),
          pl.BlockSpec((NC, blk, 16), lambda i: (0, i, 0)),
          pl.BlockSpec((blk, d), lambda i: (i, 0)),
          pl.BlockSpec((1, d), lambda i: (0, 0)),
      ],
      out_specs=[
          pl.BlockSpec((blk, d), lambda i: (i, 0)),
          pl.BlockSpec((blk, d), lambda i: (i, 0)),
      ],
      out_shape=[
          jax.ShapeDtypeStruct((n, d), jnp.float32),
          jax.ShapeDtypeStruct((n, d), jnp.float32),
      ],
  )(acc, deg, mem, b)


def kernel(x, edge_index, mem, W, b):
  n, d_in = x.shape
  d_out = W.shape[1]
  e = edge_index.shape[1]

  grain = TILES * CH * 2
  ep = ((e + grain - 1) // grain) * grain
  nch = ep // (TILES * CH)
  rows = ((n // NS) // 8 + 1) * 8
  np_ = rows * NS
  assert np_ > n and n % 8 == 0

  ei = edge_index.astype(jnp.int32)
  pad = ep - e
  pad_ar = jnp.arange(pad, dtype=jnp.int32)
  src_p = jnp.concatenate([ei[0], pad_ar % n])
  dst_p = jnp.concatenate([ei[1], n + pad_ar % (np_ - n)])
  src2 = src_p.reshape(TILES * nch, CH)
  dst2 = dst_p.reshape(TILES * nch, CH)
  ones16 = jnp.ones((CH, 16), jnp.float32)
  zeros16 = jnp.zeros((rows, 16), jnp.float32)
  zeros_d = jnp.zeros((rows, d_out), jnp.float32)

  deg = _make_deg_kernel(np_, nch)(dst2, ones16, zeros16)
  y = _mm_scale_call(x, W, deg)
  acc = _make_edge_kernel(n, np_, d_out, nch)(y, src2, dst2, zeros_d)
  spk, mem_new = _final_call(acc, deg, mem, b.reshape(1, d_out))
  return (spk, mem_new)


# speedup vs baseline: 1.1640x; 1.1450x over previous
import functools

import jax
import jax.numpy as jnp
from jax import lax
from jax.experimental import pallas as pl
from jax.experimental.pallas import tpu as pltpu
from jax.experimental.pallas import tpu_sc as plsc

BETA = 0.95
THRESH = 1.0

NC = 2
NS = 16
TILES = NC * NS
CH = 96


def _sc_mesh():
  return plsc.VectorSubcoreMesh(core_axis_name="c", subcore_axis_name="s")


_SC_PARAMS = pltpu.CompilerParams(use_tc_tiling_on_sc=False)


def _make_deg_kernel(np_, nch):
  rows = np_ // NS

  @functools.partial(
      pl.kernel,
      out_type=jax.ShapeDtypeStruct((NC, np_, 16), jnp.float32),
      mesh=_sc_mesh(),
      compiler_params=_SC_PARAMS,
      scratch_types=[
          pltpu.VMEM((nch, CH), jnp.int32),
          pltpu.VMEM((CH, 16), jnp.float32),
          pltpu.VMEM_SHARED((np_, 16), jnp.float32),
          pltpu.SemaphoreType.DMA,
      ],
  )
  def deg_kernel(dst_hbm, ones_hbm, zeros_hbm, out_hbm, dst_v, ones_v, deg_sp,
                 sem):
    c = lax.axis_index("c")
    s = lax.axis_index("s")
    wid = c * NS + s
    pltpu.sync_copy(dst_hbm.at[pl.ds(wid * nch, nch)], dst_v)
    pltpu.sync_copy(ones_hbm, ones_v)
    pltpu.sync_copy(zeros_hbm, deg_sp.at[pl.ds(s * rows, rows)])
    plsc.subcore_barrier()

    @pl.loop(0, nch)
    def _(j):
      pltpu.async_copy(ones_v, deg_sp.at[dst_v.at[j]], sem, add=True)

    @pl.loop(0, nch)
    def _(j):
      pltpu.make_async_copy(ones_v, deg_sp.at[dst_v.at[0]], sem).wait()

    plsc.subcore_barrier()
    pltpu.sync_copy(deg_sp.at[pl.ds(s * rows, rows)],
                    out_hbm.at[c, pl.ds(s * rows, rows)])

  return deg_kernel


def _make_edge_kernel(n, np_, d, nch):
  rows = np_ // NS
  qn = nch // 4
  assert qn % 3 == 0 and qn >= 3
  full_tiles = n // rows
  rem = n - full_tiles * rows

  @functools.partial(
      pl.kernel,
      out_type=jax.ShapeDtypeStruct((NC, np_, d), jnp.float32),
      mesh=_sc_mesh(),
      compiler_params=_SC_PARAMS,
      scratch_types=[
          pltpu.VMEM((qn, CH), jnp.int32),
          pltpu.VMEM((qn, CH), jnp.int32),
          pltpu.VMEM((CH, d), jnp.float32),
          pltpu.VMEM((CH, d), jnp.float32),
          pltpu.VMEM((CH, d), jnp.float32),
          pltpu.VMEM_SHARED((np_, d), jnp.float32),
          pltpu.SemaphoreType.DMA,
          pltpu.SemaphoreType.DMA,
      ],
  )
  def edge_kernel(y_hbm, src_hbm, dst_hbm, zeros_hbm, out_hbm,
                  src_v, dst_v, rows_a, rows_b, rows_c, acc_sp, sem, sem2):
    c = lax.axis_index("c")
    s = lax.axis_index("s")
    wid = c * NS + s

    @pl.when(jnp.logical_and(c == 0, s < full_tiles))
    def _():
      pltpu.sync_copy(y_hbm.at[pl.ds(s * rows, rows)],
                      acc_sp.at[pl.ds(s * rows, rows)])

    if rem > 0:
      @pl.when(jnp.logical_and(c == 0, s == full_tiles))
      def _():
        pltpu.sync_copy(y_hbm.at[pl.ds(full_tiles * rows, rem)],
                        acc_sp.at[pl.ds(full_tiles * rows, rem)])

    @pl.when(c != 0)
    def _():
      pltpu.sync_copy(zeros_hbm, acc_sp.at[pl.ds(s * rows, rows)])

    plsc.subcore_barrier()

    bufs = (rows_a, rows_b, rows_c)
    for h in range(4):
      base = wid * nch + h * qn
      pltpu.sync_copy(src_hbm.at[pl.ds(base, qn)], src_v)
      pltpu.sync_copy(dst_hbm.at[pl.ds(base, qn)], dst_v)
      pltpu.async_copy(y_hbm.at[src_v.at[0]], rows_a, sem)
      pltpu.async_copy(y_hbm.at[src_v.at[1]], rows_b, sem)

      @pl.loop(0, qn, step=3)
      def _(j):
        for k in range(3):
          cur = bufs[k]
          nxt2 = bufs[(k + 2) % 3]
          jj = j + k
          pltpu.make_async_copy(y_hbm.at[src_v.at[jj]], cur, sem).wait()

          @pl.when(jj + 2 < qn)
          def _():
            @pl.when(jj >= 1)
            def _():
              pltpu.make_async_copy(
                  nxt2, acc_sp.at[dst_v.at[jj]], sem2).wait()

            pltpu.async_copy(y_hbm.at[src_v.at[jj + 2]], nxt2, sem)

          pltpu.async_copy(cur, acc_sp.at[dst_v.at[jj]], sem2, add=True)

      pltpu.make_async_copy(rows_a, acc_sp.at[dst_v.at[0]], sem2).wait()
      pltpu.make_async_copy(rows_b, acc_sp.at[dst_v.at[0]], sem2).wait()
      pltpu.make_async_copy(rows_c, acc_sp.at[dst_v.at[0]], sem2).wait()

    plsc.subcore_barrier()
    pltpu.sync_copy(acc_sp.at[pl.ds(s * rows, rows)],
                    out_hbm.at[c, pl.ds(s * rows, rows)])

  return edge_kernel


def _mm_scale_body(x_ref, w_ref, deg_ref, y_ref):
  xw = jnp.dot(x_ref[...], w_ref[...], preferred_element_type=jnp.float32)
  deg = deg_ref[0, :, :1] + deg_ref[1, :, :1] + 1.0
  y_ref[...] = xw * lax.rsqrt(deg)


def _mm_scale_call(x, w, deg):
  n, d_in = x.shape
  d_out = w.shape[1]
  blk = 1000
  return pl.pallas_call(
      _mm_scale_body,
      grid=(n // blk,),
      in_specs=[
          pl.BlockSpec((blk, d_in), lambda i: (i, 0)),
          pl.BlockSpec((d_in, d_out), lambda i: (0, 0)),
          pl.BlockSpec((NC, blk, 16), lambda i: (0, i, 0)),
      ],
      out_specs=pl.BlockSpec((blk, d_out), lambda i: (i, 0)),
      out_shape=jax.ShapeDtypeStruct((n, d_out), jnp.float32),
  )(x, w, deg)


def _final_body(acc_ref, deg_ref, mem_ref, b_ref, spk_ref, mem_out_ref):
  deg = deg_ref[0, :, :1] + deg_ref[1, :, :1] + 1.0
  dis = lax.rsqrt(deg)
  out = (acc_ref[0] + acc_ref[1]) * dis + b_ref[...]
  mem = mem_ref[...]
  reset = (mem - THRESH > 0).astype(jnp.float32)
  mem_new = BETA * mem + out - reset * THRESH
  spk_ref[...] = (mem_new - THRESH > 0).astype(jnp.float32)
  mem_out_ref[...] = mem_new


def _final_call(acc, deg, mem, b):
  n, d = mem.shape
  blk = 1000
  grid = (n // blk,)
  return pl.pallas_call(
      _final_body,
      grid=grid,
      in_specs=[
          pl.BlockSpec((NC, blk, d), lambda i: (0, i, 0)),
          pl.BlockSpec((NC, blk, 16), lambda i: (0, i, 0)),
          pl.BlockSpec((blk, d), lambda i: (i, 0)),
          pl.BlockSpec((1, d), lambda i: (0, 0)),
      ],
      out_specs=[
          pl.BlockSpec((blk, d), lambda i: (i, 0)),
          pl.BlockSpec((blk, d), lambda i: (i, 0)),
      ],
      out_shape=[
          jax.ShapeDtypeStruct((n, d), jnp.float32),
          jax.ShapeDtypeStruct((n, d), jnp.float32),
      ],
  )(acc, deg, mem, b)


def kernel(x, edge_index, mem, W, b):
  n, d_in = x.shape
  d_out = W.shape[1]
  e = edge_index.shape[1]

  grain = TILES * CH * 12
  ep = ((e + grain - 1) // grain) * grain
  nch = ep // (TILES * CH)
  rows = ((n // NS) // 8 + 1) * 8
  np_ = rows * NS
  assert np_ > n and n % 8 == 0

  ei = edge_index.astype(jnp.int32)
  pad = ep - e
  pad_ar = jnp.arange(pad, dtype=jnp.int32)
  src_p = jnp.concatenate([ei[0], pad_ar % n])
  dst_p = jnp.concatenate([ei[1], n + pad_ar % (np_ - n)])
  src2 = src_p.reshape(TILES * nch, CH)
  dst2 = dst_p.reshape(TILES * nch, CH)
  ones16 = jnp.ones((CH, 16), jnp.float32)
  zeros16 = jnp.zeros((rows, 16), jnp.float32)
  zeros_d = jnp.zeros((rows, d_out), jnp.float32)

  deg = _make_deg_kernel(np_, nch)(dst2, ones16, zeros16)
  y = _mm_scale_call(x, W, deg)
  acc = _make_edge_kernel(n, np_, d_out, nch)(y, src2, dst2, zeros_d)
  spk, mem_new = _final_call(acc, deg, mem, b.reshape(1, d_out))
  return (spk, mem_new)


# speedup vs baseline: 1.2022x; 1.0328x over previous
import functools

import jax
import jax.numpy as jnp
from jax import lax
from jax.experimental import pallas as pl
from jax.experimental.pallas import tpu as pltpu
from jax.experimental.pallas import tpu_sc as plsc

BETA = 0.95
THRESH = 1.0

NC = 2
NS = 16
TILES = NC * NS
CH = 64
NBUF = 4


def _sc_mesh():
  return plsc.VectorSubcoreMesh(core_axis_name="c", subcore_axis_name="s")


_SC_PARAMS = pltpu.CompilerParams(use_tc_tiling_on_sc=False)


def _make_deg_kernel(np_, nch):
  rows = np_ // NS

  @functools.partial(
      pl.kernel,
      out_type=jax.ShapeDtypeStruct((NC, np_, 16), jnp.float32),
      mesh=_sc_mesh(),
      compiler_params=_SC_PARAMS,
      scratch_types=[
          pltpu.VMEM((nch, CH), jnp.int32),
          pltpu.VMEM((CH, 16), jnp.float32),
          pltpu.VMEM_SHARED((np_, 16), jnp.float32),
          pltpu.SemaphoreType.DMA,
      ],
  )
  def deg_kernel(dst_hbm, ones_hbm, zeros_hbm, out_hbm, dst_v, ones_v, deg_sp,
                 sem):
    c = lax.axis_index("c")
    s = lax.axis_index("s")
    wid = c * NS + s
    pltpu.sync_copy(dst_hbm.at[pl.ds(wid * nch, nch)], dst_v)
    pltpu.sync_copy(ones_hbm, ones_v)
    pltpu.sync_copy(zeros_hbm, deg_sp.at[pl.ds(s * rows, rows)])
    plsc.subcore_barrier()

    @pl.loop(0, nch)
    def _(j):
      pltpu.async_copy(ones_v, deg_sp.at[dst_v.at[j]], sem, add=True)

    @pl.loop(0, nch)
    def _(j):
      pltpu.make_async_copy(ones_v, deg_sp.at[dst_v.at[0]], sem).wait()

    plsc.subcore_barrier()
    pltpu.sync_copy(deg_sp.at[pl.ds(s * rows, rows)],
                    out_hbm.at[c, pl.ds(s * rows, rows)])

  return deg_kernel


def _make_edge_kernel(n, np_, d, nch):
  rows = np_ // NS
  qn = nch // 4
  assert qn % NBUF == 0 and qn >= NBUF
  full_tiles = n // rows
  rem = n - full_tiles * rows

  @functools.partial(
      pl.kernel,
      out_type=jax.ShapeDtypeStruct((NC, np_, d), jnp.float32),
      mesh=_sc_mesh(),
      compiler_params=_SC_PARAMS,
      scratch_types=[
          pltpu.VMEM((qn, CH), jnp.int32),
          pltpu.VMEM((qn, CH), jnp.int32),
      ] + [pltpu.VMEM((CH, d), jnp.float32) for _ in range(NBUF)] + [
          pltpu.VMEM_SHARED((np_, d), jnp.float32),
          pltpu.SemaphoreType.DMA,
          pltpu.SemaphoreType.DMA,
      ],
  )
  def edge_kernel(y_hbm, src_hbm, dst_hbm, zeros_hbm, out_hbm,
                  src_v, dst_v, *rest):
    bufs = rest[:NBUF]
    acc_sp, sem, sem2 = rest[NBUF:]
    c = lax.axis_index("c")
    s = lax.axis_index("s")
    wid = c * NS + s

    @pl.when(jnp.logical_and(c == 0, s < full_tiles))
    def _():
      pltpu.sync_copy(y_hbm.at[pl.ds(s * rows, rows)],
                      acc_sp.at[pl.ds(s * rows, rows)])

    if rem > 0:
      @pl.when(jnp.logical_and(c == 0, s == full_tiles))
      def _():
        pltpu.sync_copy(y_hbm.at[pl.ds(full_tiles * rows, rem)],
                        acc_sp.at[pl.ds(full_tiles * rows, rem)])

    @pl.when(c != 0)
    def _():
      pltpu.sync_copy(zeros_hbm, acc_sp.at[pl.ds(s * rows, rows)])

    plsc.subcore_barrier()

    lead = NBUF - 1
    for h in range(4):
      base = wid * nch + h * qn
      pltpu.sync_copy(src_hbm.at[pl.ds(base, qn)], src_v)
      pltpu.sync_copy(dst_hbm.at[pl.ds(base, qn)], dst_v)
      for p in range(lead):
        pltpu.async_copy(y_hbm.at[src_v.at[p]], bufs[p], sem)

      @pl.loop(0, qn, step=NBUF)
      def _(j):
        for k in range(NBUF):
          cur = bufs[k]
          nxt = bufs[(k + lead) % NBUF]
          jj = j + k
          pltpu.make_async_copy(y_hbm.at[src_v.at[jj]], cur, sem).wait()

          @pl.when(jj + lead < qn)
          def _():
            @pl.when(jj >= 1)
            def _():
              pltpu.make_async_copy(
                  nxt, acc_sp.at[dst_v.at[jj]], sem2).wait()

            pltpu.async_copy(y_hbm.at[src_v.at[jj + lead]], nxt, sem)

          pltpu.async_copy(cur, acc_sp.at[dst_v.at[jj]], sem2, add=True)

      for p in range(NBUF):
        pltpu.make_async_copy(bufs[p], acc_sp.at[dst_v.at[0]], sem2).wait()

    plsc.subcore_barrier()
    pltpu.sync_copy(acc_sp.at[pl.ds(s * rows, rows)],
                    out_hbm.at[c, pl.ds(s * rows, rows)])

  return edge_kernel


def _mm_scale_body(x_ref, w_ref, deg_ref, y_ref):
  xw = jnp.dot(x_ref[...], w_ref[...], preferred_element_type=jnp.float32)
  deg = deg_ref[0, :, :1] + deg_ref[1, :, :1] + 1.0
  y_ref[...] = xw * lax.rsqrt(deg)


def _mm_scale_call(x, w, deg):
  n, d_in = x.shape
  d_out = w.shape[1]
  blk = 1000
  return pl.pallas_call(
      _mm_scale_body,
      grid=(n // blk,),
      in_specs=[
          pl.BlockSpec((blk, d_in), lambda i: (i, 0)),
          pl.BlockSpec((d_in, d_out), lambda i: (0, 0)),
          pl.BlockSpec((NC, blk, 16), lambda i: (0, i, 0)),
      ],
      out_specs=pl.BlockSpec((blk, d_out), lambda i: (i, 0)),
      out_shape=jax.ShapeDtypeStruct((n, d_out), jnp.float32),
  )(x, w, deg)


def _final_body(acc_ref, deg_ref, mem_ref, b_ref, spk_ref, mem_out_ref):
  deg = deg_ref[0, :, :1] + deg_ref[1, :, :1] + 1.0
  dis = lax.rsqrt(deg)
  out = (acc_ref[0] + acc_ref[1]) * dis + b_ref[...]
  mem = mem_ref[...]
  reset = (mem - THRESH > 0).astype(jnp.float32)
  mem_new = BETA * mem + out - reset * THRESH
  spk_ref[...] = (mem_new - THRESH > 0).astype(jnp.float32)
  mem_out_ref[...] = mem_new


def _final_call(acc, deg, mem, b):
  n, d = mem.shape
  blk = 1000
  grid = (n // blk,)
  return pl.pallas_call(
      _final_body,
      grid=grid,
      in_specs=[
          pl.BlockSpec((NC, blk, d), lambda i: (0, i, 0)),
          pl.BlockSpec((NC, blk, 16), lambda i: (0, i, 0)),
          pl.BlockSpec((blk, d), lambda i: (i, 0)),
          pl.BlockSpec((1, d), lambda i: (0, 0)),
      ],
      out_specs=[
          pl.BlockSpec((blk, d), lambda i: (i, 0)),
          pl.BlockSpec((blk, d), lambda i: (i, 0)),
      ],
      out_shape=[
          jax.ShapeDtypeStruct((n, d), jnp.float32),
          jax.ShapeDtypeStruct((n, d), jnp.float32),
      ],
  )(acc, deg, mem, b)


def kernel(x, edge_index, mem, W, b):
  n, d_in = x.shape
  d_out = W.shape[1]
  e = edge_index.shape[1]

  grain = TILES * CH * 4 * NBUF
  ep = ((e + grain - 1) // grain) * grain
  nch = ep // (TILES * CH)
  rows = ((n // NS) // 8 + 1) * 8
  np_ = rows * NS
  assert np_ > n and n % 8 == 0

  ei = edge_index.astype(jnp.int32)
  pad = ep - e
  pad_ar = jnp.arange(pad, dtype=jnp.int32)
  src_p = jnp.concatenate([ei[0], pad_ar % n])
  dst_p = jnp.concatenate([ei[1], n + pad_ar % (np_ - n)])
  src2 = src_p.reshape(TILES * nch, CH)
  dst2 = dst_p.reshape(TILES * nch, CH)
  ones16 = jnp.ones((CH, 16), jnp.float32)
  zeros16 = jnp.zeros((rows, 16), jnp.float32)
  zeros_d = jnp.zeros((rows, d_out), jnp.float32)

  deg = _make_deg_kernel(np_, nch)(dst2, ones16, zeros16)
  y = _mm_scale_call(x, W, deg)
  acc = _make_edge_kernel(n, np_, d_out, nch)(y, src2, dst2, zeros_d)
  spk, mem_new = _final_call(acc, deg, mem, b.reshape(1, d_out))
  return (spk, mem_new)
